# baseline (speedup 1.0000x reference)
import functools

import jax
import jax.numpy as jnp
from jax import lax
from jax.experimental import pallas as pl
from jax.experimental.pallas import tpu as pltpu
from jax.experimental.pallas import tpu_sc as plsc

B = 16384
MET_D = 4
OUT_D = 13
TABLE_N = 10
EMB_DIM = 8

NUM_CORES = 1
NUM_SUBCORES = 16
LANES = 16
NUM_WORKERS = NUM_CORES * NUM_SUBCORES
ROWS_PER_W = B // NUM_WORKERS
GROUPS = ROWS_PER_W // LANES

HALF_PI = 1.5707963267948966
PI = 3.141592653589793

ATAN_C = (0.99997726, -0.33262347, 0.19354346,
          -0.11643287, 0.05265332, -0.01172120)


def _rsqrt(a):
    i = lax.bitcast_convert_type(a, jnp.int32)
    i = jnp.int32(0x5F3759DF) - lax.shift_right_logical(i, 1)
    y = lax.bitcast_convert_type(i, jnp.float32)
    for _ in range(2):
        y = y * (1.5 - 0.5 * a * y * y)
    return y


def _atan2(y, x):
    ax = jnp.abs(x)
    ay = jnp.abs(y)
    hi = jnp.maximum(ax, ay)
    lo = jnp.minimum(ax, ay)
    t = lo / jnp.maximum(hi, 1e-30)
    t2 = t * t
    p = jnp.float32(ATAN_C[5])
    for k in (4, 3, 2, 1, 0):
        p = p * t2 + ATAN_C[k]
    p = p * t
    p = jnp.where(ay > ax, HALF_PI - p, p)
    p = jnp.where(x < 0, PI - p, p)
    return jnp.where(y < 0, -p, p)


@functools.partial(
    pl.kernel,
    out_type=jax.ShapeDtypeStruct((B, OUT_D), jnp.float32),
    mesh=plsc.VectorSubcoreMesh(core_axis_name="c", subcore_axis_name="s",
                                num_cores=NUM_CORES),
    compiler_params=pltpu.CompilerParams(
        use_tc_tiling_on_sc=False, needs_layout_passes=False,
        skip_device_barrier=True),
    scratch_types=[
        pltpu.VMEM((ROWS_PER_W, MET_D), jnp.float32),
        pltpu.VMEM((TABLE_N, EMB_DIM), jnp.float32),
        pltpu.VMEM((ROWS_PER_W, OUT_D), jnp.float32),
    ],
)
def _process_metrics_sc(metrics_hbm, emb_hbm, out_hbm, met_v, emb_v, out_v):
    wid = lax.axis_index("s") * NUM_CORES + lax.axis_index("c")
    base = wid * ROWS_PER_W
    pltpu.sync_copy(metrics_hbm.at[pl.ds(base, ROWS_PER_W)], met_v)
    pltpu.sync_copy(emb_hbm, emb_v)
    iota = lax.iota(jnp.int32, LANES)
    col_idx = [jnp.full((LANES,), c, jnp.int32) for c in range(OUT_D)]

    @plsc.parallel_loop(0, GROUPS, step=1, unroll=8)
    def _group(g):
        rows = g * LANES + iota

        def getcol(c):
            return plsc.load_gather(met_v, [rows, col_idx[c]])

        def putcol(c, v):
            plsc.store_scatter(out_v, [rows, col_idx[c]], v)

        x = getcol(0)
        y = getcol(1)
        sp = getcol(2)
        rof = getcol(3)

        r2 = x * x + y * y
        r2c = jnp.maximum(r2, 1e-30)
        r = r2c * _rsqrt(r2c)
        theta = _atan2(y, x)

        putcol(0, 1000.0 * x)
        putcol(1, 1000.0 * y)
        putcol(2, 1000.0 * r)
        putcol(3, 0.3 * theta)
        putcol(4, 0.1 * sp)

        ro = rof.astype(jnp.int32)
        for d in range(EMB_DIM):
            v = plsc.load_gather(emb_v, [ro, col_idx[d]])
            putcol(5 + d, v)

    pltpu.sync_copy(out_v, out_hbm.at[pl.ds(base, ROWS_PER_W)])


def kernel(metrics, emb_table):
    out = _process_metrics_sc(metrics, emb_table)
    return (out, out)


# speedup vs baseline: 1.0190x; 1.0190x over previous
import functools

import jax
import jax.numpy as jnp
from jax import lax
from jax.experimental import pallas as pl
from jax.experimental.pallas import tpu as pltpu
from jax.experimental.pallas import tpu_sc as plsc

B = 16384
MET_D = 4
OUT_D = 13
TABLE_N = 10
EMB_DIM = 8

NUM_CORES = 1
NUM_SUBCORES = 16
LANES = 16
NUM_WORKERS = NUM_CORES * NUM_SUBCORES
ROWS_PER_W = B // NUM_WORKERS
GROUPS = ROWS_PER_W // LANES

HALF_PI = 1.5707963267948966
PI = 3.141592653589793

ATAN_C = (0.99997726, -0.33262347, 0.19354346,
          -0.11643287, 0.05265332, -0.01172120)


def _rsqrt(a):
    i = lax.bitcast_convert_type(a, jnp.int32)
    i = jnp.int32(0x5F3759DF) - lax.shift_right_logical(i, 1)
    y = lax.bitcast_convert_type(i, jnp.float32)
    for _ in range(2):
        y = y * (1.5 - 0.5 * a * y * y)
    return y


def _atan2(y, x):
    ax = jnp.abs(x)
    ay = jnp.abs(y)
    hi = jnp.maximum(ax, ay)
    lo = jnp.minimum(ax, ay)
    t = lo / jnp.maximum(hi, 1e-30)
    t2 = t * t
    p = jnp.float32(ATAN_C[5])
    for k in (4, 3, 2, 1, 0):
        p = p * t2 + ATAN_C[k]
    p = p * t
    p = jnp.where(ay > ax, HALF_PI - p, p)
    p = jnp.where(x < 0, PI - p, p)
    return jnp.where(y < 0, -p, p)


@functools.partial(
    pl.kernel,
    out_type=jax.ShapeDtypeStruct((B, OUT_D), jnp.float32),
    mesh=plsc.VectorSubcoreMesh(core_axis_name="c", subcore_axis_name="s",
                                num_cores=NUM_CORES),
    compiler_params=pltpu.CompilerParams(
        use_tc_tiling_on_sc=False, needs_layout_passes=False,
        skip_device_barrier=True),
    scratch_types=[
        pltpu.VMEM((ROWS_PER_W // 2, MET_D), jnp.float32),
        pltpu.VMEM((ROWS_PER_W // 2, MET_D), jnp.float32),
        pltpu.VMEM((TABLE_N, EMB_DIM), jnp.float32),
        pltpu.VMEM((ROWS_PER_W // 2, OUT_D), jnp.float32),
        pltpu.VMEM((ROWS_PER_W // 2, OUT_D), jnp.float32),
        pltpu.SemaphoreType.DMA,
        pltpu.SemaphoreType.DMA,
        pltpu.SemaphoreType.DMA,
    ],
)
def _process_metrics_sc(metrics_hbm, emb_hbm, out_hbm,
                        met0, met1, emb_v, out0, out1,
                        s_in0, s_in1, s_out):
    half = ROWS_PER_W // 2
    hgroups = half // LANES
    wid = lax.axis_index("s") * NUM_CORES + lax.axis_index("c")
    base = wid * ROWS_PER_W
    in0 = pltpu.async_copy(metrics_hbm.at[pl.ds(base, half)], met0, s_in0)
    in1 = pltpu.async_copy(metrics_hbm.at[pl.ds(base + half, half)],
                           met1, s_in1)
    pltpu.sync_copy(emb_hbm, emb_v)
    iota = lax.iota(jnp.int32, LANES)
    col_idx = [jnp.full((LANES,), c, jnp.int32) for c in range(OUT_D)]

    def compute_half(met_v, out_v):
        @plsc.parallel_loop(0, hgroups, step=1, unroll=4)
        def _group(g):
            rows = g * LANES + iota

            def getcol(c):
                return plsc.load_gather(met_v, [rows, col_idx[c]])

            def putcol(c, v):
                plsc.store_scatter(out_v, [rows, col_idx[c]], v)

            x = getcol(0)
            y = getcol(1)
            sp = getcol(2)
            rof = getcol(3)

            r2 = x * x + y * y
            r2c = jnp.maximum(r2, 1e-30)
            r = r2c * _rsqrt(r2c)
            theta = _atan2(y, x)

            putcol(0, 1000.0 * x)
            putcol(1, 1000.0 * y)
            putcol(2, 1000.0 * r)
            putcol(3, 0.3 * theta)
            putcol(4, 0.1 * sp)

            ro = rof.astype(jnp.int32)
            for d in range(EMB_DIM):
                v = plsc.load_gather(emb_v, [ro, col_idx[d]])
                putcol(5 + d, v)

    in0.wait()
    compute_half(met0, out0)
    o0 = pltpu.async_copy(out0, out_hbm.at[pl.ds(base, half)], s_out)
    in1.wait()
    compute_half(met1, out1)
    o0.wait()
    pltpu.sync_copy(out1, out_hbm.at[pl.ds(base + half, half)])


def kernel(metrics, emb_table):
    out = _process_metrics_sc(metrics, emb_table)
    return (out, out)
